# baseline (speedup 1.0000x reference)
import functools

import jax
import jax.numpy as jnp
from jax.experimental import pallas as pl
from jax.experimental.pallas import tpu as pltpu

GROUP = 8
EPS = 1e-05
BLK = 512


def _pass1(x_ref, w1_ref, b1_ref, w2_ref, b2_ref, stats_ref, oh_ref, w12_ref):
    i = pl.program_id(0)

    @pl.when(i == 0)
    def _():
        w12_ref[...] = jnp.dot(w1_ref[...], w2_ref[...],
                               preferred_element_type=jnp.float32)
        stats_ref[...] = jnp.zeros_like(stats_ref)

    xb = x_ref[...]
    b12 = jnp.dot(b1_ref[...], w2_ref[...],
                  preferred_element_type=jnp.float32) + b2_ref[...]
    logits = jnp.dot(xb, w12_ref[...],
                     preferred_element_type=jnp.float32) + b12
    mx = jnp.max(logits, axis=1, keepdims=True)
    colid = jax.lax.broadcasted_iota(jnp.int32, logits.shape, 1)
    idx = jnp.min(jnp.where(logits >= mx, colid, GROUP), axis=1, keepdims=True)
    oh = (colid == idx).astype(jnp.float32)
    oh_ref[...] = oh

    rs = jnp.sum(xb, axis=1, keepdims=True)
    rss = jnp.sum(xb * xb, axis=1, keepdims=True)
    cnt_g = jnp.sum(oh, axis=0)
    sum_g = jnp.sum(oh * rs, axis=0)
    ssq_g = jnp.sum(oh * rss, axis=0)
    stats_ref[...] += jnp.concatenate(
        [cnt_g[None, :], sum_g[None, :], ssq_g[None, :]], axis=0)


def _pass2(hw, x_ref, oh_ref, stats_ref, w_ref, b_ref, out_ref):
    cnt_rows = stats_ref[0:1, :]
    total = cnt_rows * float(hw)
    s = stats_ref[1:2, :]
    q = stats_ref[2:3, :]
    mean = s / jnp.maximum(total, 1.0)
    sq = q - s * mean
    var = sq / jnp.maximum(total - 1.0, 1.0)
    rstd = jax.lax.rsqrt(var + EPS)

    oh = oh_ref[...]
    mean_r = jnp.sum(oh * mean, axis=1, keepdims=True)
    rstd_r = jnp.sum(oh * rstd, axis=1, keepdims=True)
    out_ref[...] = ((x_ref[...] - mean_r) * rstd_r) * w_ref[...] + b_ref[...]


def kernel(x, W1, b1, W2, b2, weight, bias):
    n, c, h, w = x.shape
    hw = h * w
    r = n * c
    nb = r // BLK
    x2 = x.reshape(r, hw)
    wrow = jnp.broadcast_to(weight.reshape(1, c), (n, c)).reshape(r, 1)
    brow = jnp.broadcast_to(bias.reshape(1, c), (n, c)).reshape(r, 1)

    stats, oh = pl.pallas_call(
        _pass1,
        grid=(nb,),
        in_specs=[
            pl.BlockSpec((BLK, hw), lambda i: (i, 0)),
            pl.BlockSpec((hw, hw), lambda i: (0, 0)),
            pl.BlockSpec((1, hw), lambda i: (0, 0)),
            pl.BlockSpec((hw, GROUP), lambda i: (0, 0)),
            pl.BlockSpec((1, GROUP), lambda i: (0, 0)),
        ],
        out_specs=[
            pl.BlockSpec((3, GROUP), lambda i: (0, 0)),
            pl.BlockSpec((BLK, GROUP), lambda i: (i, 0)),
        ],
        out_shape=[
            jax.ShapeDtypeStruct((3, GROUP), jnp.float32),
            jax.ShapeDtypeStruct((r, GROUP), jnp.float32),
        ],
        scratch_shapes=[pltpu.VMEM((hw, GROUP), jnp.float32)],
    )(x2, W1, b1.reshape(1, hw), W2, b2.reshape(1, GROUP))

    out2 = pl.pallas_call(
        functools.partial(_pass2, hw),
        grid=(nb,),
        in_specs=[
            pl.BlockSpec((BLK, hw), lambda i: (i, 0)),
            pl.BlockSpec((BLK, GROUP), lambda i: (i, 0)),
            pl.BlockSpec((3, GROUP), lambda i: (0, 0)),
            pl.BlockSpec((BLK, 1), lambda i: (i, 0)),
            pl.BlockSpec((BLK, 1), lambda i: (i, 0)),
        ],
        out_specs=pl.BlockSpec((BLK, hw), lambda i: (i, 0)),
        out_shape=jax.ShapeDtypeStruct((r, hw), jnp.float32),
    )(x2, oh, stats, wrow, brow)

    return out2.reshape(n, c, h, w)


# speedup vs baseline: 4.7023x; 4.7023x over previous
import functools

import jax
import jax.numpy as jnp
from jax.experimental import pallas as pl
from jax.experimental.pallas import tpu as pltpu

GROUP = 8
EPS = 1e-05
HB = 512


def _pass1(x_ref, w1_ref, b1_ref, b2_ref, w2_ref, stats_ref, oh_ref, w12t_ref):
    i = pl.program_id(0)

    @pl.when(i == 0)
    def _():
        w12t_ref[...] = jax.lax.dot_general(
            w2_ref[...], w1_ref[...],
            (((0,), (1,)), ((), ())),
            preferred_element_type=jnp.float32)
        stats_ref[...] = jnp.zeros_like(stats_ref)

    xb = x_ref[0]
    b12 = jnp.sum(w12t_ref[...] * b1_ref[...], axis=1,
                  keepdims=True) + b2_ref[...]
    lt = jnp.dot(w12t_ref[...], xb,
                 preferred_element_type=jnp.float32) + b12
    mx = jnp.max(lt, axis=0, keepdims=True)
    rowid = jax.lax.broadcasted_iota(jnp.int32, lt.shape, 0)
    idx = jnp.min(jnp.where(lt >= mx, rowid, GROUP), axis=0, keepdims=True)
    oh = (rowid == idx).astype(jnp.float32)
    oh_ref[0] = oh

    csum = jnp.sum(xb, axis=0, keepdims=True)
    cssq = jnp.sum(xb * xb, axis=0, keepdims=True)
    cnt_g = jnp.sum(oh, axis=1, keepdims=True)
    sum_g = jnp.sum(oh * csum, axis=1, keepdims=True)
    ssq_g = jnp.sum(oh * cssq, axis=1, keepdims=True)
    stats_ref[...] += jnp.concatenate([cnt_g, sum_g, ssq_g], axis=1)


def _pass2(hw, x_ref, oh_ref, stats_ref, w_ref, b_ref, out_ref):
    cnt_rows = stats_ref[:, 0:1]
    total = cnt_rows * float(hw)
    s = stats_ref[:, 1:2]
    q = stats_ref[:, 2:3]
    mean = s / jnp.maximum(total, 1.0)
    sq = q - s * mean
    var = sq / jnp.maximum(total - 1.0, 1.0)
    rstd = jax.lax.rsqrt(var + EPS)

    oh = oh_ref[0]
    mean_c = jnp.sum(oh * mean, axis=0, keepdims=True)
    rstd_c = jnp.sum(oh * rstd, axis=0, keepdims=True)
    out_ref[...] = ((x_ref[...] - mean_c) * rstd_c) * w_ref[...] + b_ref[...]


def kernel(x, W1, b1, W2, b2, weight, bias):
    n, c, h, w = x.shape
    hw = h * w
    nb2 = hw // HB
    xt = jnp.transpose(x, (0, 2, 3, 1)).reshape(n * hw, c)

    stats, oh = pl.pallas_call(
        _pass1,
        grid=(n,),
        in_specs=[
            pl.BlockSpec((1, hw, c), lambda i: (i, 0, 0)),
            pl.BlockSpec((hw, hw), lambda i: (0, 0)),
            pl.BlockSpec((1, hw), lambda i: (0, 0)),
            pl.BlockSpec((GROUP, 1), lambda i: (0, 0)),
            pl.BlockSpec((hw, GROUP), lambda i: (0, 0)),
        ],
        out_specs=[
            pl.BlockSpec((GROUP, 3), lambda i: (0, 0)),
            pl.BlockSpec((1, GROUP, c), lambda i: (i, 0, 0)),
        ],
        out_shape=[
            jax.ShapeDtypeStruct((GROUP, 3), jnp.float32),
            jax.ShapeDtypeStruct((n, GROUP, c), jnp.float32),
        ],
        scratch_shapes=[pltpu.VMEM((GROUP, hw), jnp.float32)],
    )(xt.reshape(n, hw, c), W1, b1.reshape(1, hw), b2.reshape(GROUP, 1), W2)

    out2 = pl.pallas_call(
        functools.partial(_pass2, hw),
        grid=(n, nb2),
        in_specs=[
            pl.BlockSpec((HB, c), lambda i, j: (i * nb2 + j, 0)),
            pl.BlockSpec((1, GROUP, c), lambda i, j: (i, 0, 0)),
            pl.BlockSpec((GROUP, 3), lambda i, j: (0, 0)),
            pl.BlockSpec((1, c), lambda i, j: (0, 0)),
            pl.BlockSpec((1, c), lambda i, j: (0, 0)),
        ],
        out_specs=pl.BlockSpec((HB, c), lambda i, j: (i * nb2 + j, 0)),
        out_shape=jax.ShapeDtypeStruct((n * hw, c), jnp.float32),
    )(xt, oh, stats, weight.reshape(1, c), bias.reshape(1, c))

    return jnp.transpose(out2.reshape(n, h, w, c), (0, 3, 1, 2))


# speedup vs baseline: 7.5742x; 1.6107x over previous
import jax
import jax.numpy as jnp
from jax.experimental import pallas as pl
from jax.experimental.pallas import tpu as pltpu

GROUP = 8
EPS = 1e-05


def _fused(x_ref, w1_ref, b1_ref, b2_ref, w2_ref, w_ref, b_ref, out_ref,
           w12t_ref, cache_ref, oh_ref, stats_ref, scale_ref, off_ref):
    p = pl.program_id(0)
    i = pl.program_id(1)
    n = pl.num_programs(1)
    hw = x_ref.shape[1]

    @pl.when((p == 0) & (i == 0))
    def _():
        w12t_ref[...] = jax.lax.dot_general(
            w2_ref[...], w1_ref[...],
            (((0,), (1,)), ((), ())),
            preferred_element_type=jnp.float32)
        stats_ref[...] = jnp.zeros_like(stats_ref)

    @pl.when(p == 0)
    def _():
        xb = x_ref[0]
        cache_ref[i] = xb
        b12 = jnp.sum(w12t_ref[...] * b1_ref[...], axis=1,
                      keepdims=True) + b2_ref[...]
        lt = jnp.dot(w12t_ref[...], xb,
                     preferred_element_type=jnp.float32) + b12
        mx = jnp.max(lt, axis=0, keepdims=True)
        rowid = jax.lax.broadcasted_iota(jnp.int32, lt.shape, 0)
        idx = jnp.min(jnp.where(lt >= mx, rowid, GROUP), axis=0, keepdims=True)
        oh = (rowid == idx).astype(jnp.float32)
        oh_ref[i] = oh

        csum = jnp.sum(xb, axis=0, keepdims=True)
        cssq = jnp.sum(xb * xb, axis=0, keepdims=True)
        cnt_g = jnp.sum(oh, axis=1, keepdims=True)
        sum_g = jnp.sum(oh * csum, axis=1, keepdims=True)
        ssq_g = jnp.sum(oh * cssq, axis=1, keepdims=True)
        stats_ref[...] += jnp.concatenate([cnt_g, sum_g, ssq_g], axis=1)

    @pl.when((p == 1) & (i == 0))
    def _():
        cnt_rows = stats_ref[:, 0:1]
        total = cnt_rows * float(hw)
        s = stats_ref[:, 1:2]
        q = stats_ref[:, 2:3]
        mean = s / jnp.maximum(total, 1.0)
        sq = q - s * mean
        var = sq / jnp.maximum(total - 1.0, 1.0)
        rstd = jax.lax.rsqrt(var + EPS)
        for k in range(n):
            oh = oh_ref[k]
            rstd_c = jnp.sum(oh * rstd, axis=0, keepdims=True)
            mean_c = jnp.sum(oh * mean, axis=0, keepdims=True)
            sc = rstd_c * w_ref[...]
            scale_ref[k] = sc
            off_ref[k] = b_ref[...] - mean_c * sc

    @pl.when(p == 1)
    def _():
        out_ref[...] = cache_ref[i] * scale_ref[i] + off_ref[i]


def kernel(x, W1, b1, W2, b2, weight, bias):
    n, c, h, w = x.shape
    hw = h * w
    xt = jnp.transpose(x, (0, 2, 3, 1)).reshape(n, hw, c)

    out2 = pl.pallas_call(
        _fused,
        grid=(2, n),
        in_specs=[
            pl.BlockSpec((1, hw, c), lambda p, i: (jnp.where(p == 0, i, n - 1), 0, 0)),
            pl.BlockSpec((hw, hw), lambda p, i: (0, 0)),
            pl.BlockSpec((1, hw), lambda p, i: (0, 0)),
            pl.BlockSpec((GROUP, 1), lambda p, i: (0, 0)),
            pl.BlockSpec((hw, GROUP), lambda p, i: (0, 0)),
            pl.BlockSpec((1, c), lambda p, i: (0, 0)),
            pl.BlockSpec((1, c), lambda p, i: (0, 0)),
        ],
        out_specs=pl.BlockSpec((hw, c), lambda p, i: (jnp.where(p == 0, 0, i), 0)),
        out_shape=jax.ShapeDtypeStruct((n * hw, c), jnp.float32),
        scratch_shapes=[
            pltpu.VMEM((GROUP, hw), jnp.float32),
            pltpu.VMEM((n, hw, c), jnp.float32),
            pltpu.VMEM((n, GROUP, c), jnp.float32),
            pltpu.VMEM((GROUP, 3), jnp.float32),
            pltpu.VMEM((n, 1, c), jnp.float32),
            pltpu.VMEM((n, 1, c), jnp.float32),
        ],
    )(xt, W1, b1.reshape(1, hw), b2.reshape(GROUP, 1), W2,
      weight.reshape(1, c), bias.reshape(1, c))

    return jnp.transpose(out2.reshape(n, h, w, c), (0, 3, 1, 2))
